# baseline (speedup 1.0000x reference)
import functools
import math

import jax
import jax.numpy as jnp
from jax.experimental import pallas as pl
from jax.experimental.pallas import tpu as pltpu

_DT = jnp.bfloat16
_F8 = jnp.float8_e4m3fn

_TRANS_A = (((0,), (0,)), ((), ()))
_TRANS_B = (((1,), (1,)), ((), ()))


def _fused_kernel(c_ref, ei_ref, pwf_ref, lb_ref, lw_ref, x_ref, pw_ref,
                  res_ref, lap_s, g0_s, g1_s, outs_s, sacc_s, wv_s,
                  *, n, f, tf, kk):
    i = pl.program_id(0)
    nsteps = f // tf + 1

    @pl.when(i == 0)
    def _():
        e = ei_ref.shape[1]
        ids = jax.lax.broadcasted_iota(jnp.int32, (n, e), 0)
        ert = (ids == ei_ref[0:1, :]).astype(_F8)
        ect = (ids == ei_ref[1:2, :]).astype(_F8)
        a = jax.lax.dot_general(ert, ect, _TRANS_B,
                                preferred_element_type=jnp.float32)
        deg = jnp.sum(a, axis=1, keepdims=True)
        rows = jax.lax.broadcasted_iota(jnp.int32, (n, n), 0)
        cols = jax.lax.broadcasted_iota(jnp.int32, (n, n), 1)
        eye = rows == cols
        lap = jnp.where(eye, deg - a, -a)
        lap_s[...] = lap

        def compose(cv, g_s):
            h = cv[0]
            alpha = cv[1]
            off = jnp.where(eye, 0.0, lap).astype(_DT)
            ld = deg - alpha
            hld = h * ld
            den = 1.0 / (hld * hld + 1.0)
            tlr = hld * den
            tli = -den
            dr = (-h) * tlr
            di = (-h) * tli

            hlm = h * jnp.where(eye, lap - alpha, lap)
            br = tlr * hlm + jnp.where(eye, tli, 0.0)
            bi = tli * hlm - jnp.where(eye, tlr, 0.0)

            def jmul(ur, ui):
                sr = (dr * ur - di * ui).astype(_DT)
                si = (dr * ui + di * ur).astype(_DT)
                return (jax.lax.dot_general(
                            off, sr, _TRANS_A,
                            preferred_element_type=jnp.float32),
                        jax.lax.dot_general(
                            off, si, _TRANS_A,
                            preferred_element_type=jnp.float32))

            jbr, jbi = jmul(br, bi)
            mr, mi = jmul(jbr + br, jbi + bi)
            mr = mr + br
            mi = mi + bi

            mrl = mr.astype(_DT)
            mil = mi.astype(_DT)
            msl = (mr + mi).astype(_DT)
            t1 = jnp.dot(mrl, mrl, preferred_element_type=jnp.float32)
            t2 = jnp.dot(mil, mil, preferred_element_type=jnp.float32)
            t3 = jnp.dot(msl, msl, preferred_element_type=jnp.float32)
            m2r = t1 - t2
            m2i = t3 - t1 - t2

            g = (2.0 * (cv[3] * mr - cv[4] * mi)
                 + 2.0 * (cv[5] * m2r - cv[6] * m2i))
            g_s[...] = (g + jnp.where(eye, cv[2], 0.0)).astype(g_s.dtype)

        compose([c_ref[0, k] for k in range(7)], g0_s)
        compose([c_ref[1, k] for k in range(7)], g1_s)
        sacc_s[...] = jnp.zeros_like(sacc_s)

    @pl.when(i >= 1)
    def _():
        j = i - 1
        x = x_ref[...].astype(_DT)
        hid = jnp.dot(g0_s[...], x, preferred_element_type=jnp.float32)
        hid = jnp.maximum(hid, 0.0).astype(_DT)
        o = jnp.dot(g1_s[...], hid, preferred_element_type=jnp.float32)
        o = jnp.maximum(o, 0.0)
        outs_s[:, pl.ds(pl.multiple_of(j * tf, tf), tf)] = o.astype(_DT)
        sacc_s[...] += jax.lax.dot_general(
            pw_ref[...], o, (((0,), (1,)), ((), ())),
            preferred_element_type=jnp.float32)

    @pl.when(i == nsteps - 1)
    def _():
        pw = pwf_ref[...]
        inv_norm = 1.0 / jnp.sqrt(jnp.sum(pw * pw))
        score = jnp.tanh(sacc_s[...] * inv_norm)
        b = pltpu.bitcast(score, jnp.int32)
        key = jnp.where(b < 0, jnp.int32(-2147483648) - b, b)
        top = jnp.int32(0x3F800000)

        def body(_, carry):
            lo, hi = carry
            mid = lo + (hi - lo + 1) // 2
            go = jnp.sum((key >= mid).astype(jnp.int32)) >= kk
            return (jnp.where(go, mid, lo), jnp.where(go, hi, mid - 1))

        lo, _ = jax.lax.fori_loop(0, 32, body, (-top, top))

        c_gt = jnp.sum((key > lo).astype(jnp.int32))
        tie = key == lo
        ii = jax.lax.broadcasted_iota(jnp.int32, (n, n), 0)
        jj = jax.lax.broadcasted_iota(jnp.int32, (n, n), 1)
        tri = (ii <= jj).astype(_DT)
        pref = jnp.dot(tie.astype(_DT), tri,
                       preferred_element_type=jnp.float32)
        sel = (key > lo) | (tie & (pref <= (kk - c_gt).astype(jnp.float32)))
        wv_s[...] = jnp.where(sel, score, 0.0)

        vt = jnp.dot(wv_s[...].astype(_DT), outs_s[...],
                     preferred_element_type=jnp.float32) * (1.0 / kk)
        ft = jax.lax.dot_general(vt.astype(_DT), lw_ref[...].astype(_DT),
                                 _TRANS_B, preferred_element_type=jnp.float32)
        res_ref[...] = lb_ref[...] + ft


def kernel(x, edge_index, batch,
           conv0_h, conv0_alpha, conv0_c0, conv0_cjr, conv0_cji,
           conv1_h, conv1_alpha, conv1_c0, conv1_cjr, conv1_cji,
           pool_w, lin_w, lin_b):
    del batch
    n, f = x.shape
    e = edge_index.shape[1]
    nout = lin_w.shape[0]
    tf = min(512, f)
    nf = f // tf
    kk = int(math.ceil(0.9 * n))

    cvec = jnp.stack([
        jnp.stack([conv0_h, conv0_alpha, conv0_c0, conv0_cjr[0],
                   conv0_cji[0], conv0_cjr[1], conv0_cji[1]]),
        jnp.stack([conv1_h, conv1_alpha, conv1_c0, conv1_cjr[0],
                   conv1_cji[0], conv1_cjr[1], conv1_cji[1]]),
    ]).astype(jnp.float32)

    res = pl.pallas_call(
        functools.partial(_fused_kernel, n=n, f=f, tf=tf, kk=kk),
        out_shape=jax.ShapeDtypeStruct((1, nout), jnp.float32),
        grid=(nf + 1,),
        in_specs=[pl.BlockSpec(memory_space=pltpu.MemorySpace.SMEM),
                  pl.BlockSpec((2, e), lambda i: (0, 0)),
                  pl.BlockSpec((1, f), lambda i: (0, 0)),
                  pl.BlockSpec((1, nout), lambda i: (0, 0)),
                  pl.BlockSpec((nout, f), lambda i: (0, 0)),
                  pl.BlockSpec((n, tf),
                               lambda i: (0, jnp.clip(i - 1, 0, nf - 1))),
                  pl.BlockSpec((tf, 1),
                               lambda i: (jnp.clip(i - 1, 0, nf - 1), 0))],
        out_specs=pl.BlockSpec((1, nout), lambda i: (0, 0)),
        scratch_shapes=[pltpu.VMEM((n, n), jnp.float32),
                        pltpu.VMEM((n, n), _DT),
                        pltpu.VMEM((n, n), _DT),
                        pltpu.VMEM((n, f), _DT),
                        pltpu.VMEM((1, n), jnp.float32),
                        pltpu.VMEM((1, n), jnp.float32)],
        compiler_params=pltpu.CompilerParams(
            dimension_semantics=("arbitrary",)),
    )(cvec, edge_index, pool_w.reshape(1, f), lin_b.reshape(1, nout),
      lin_w, x, pool_w.reshape(f, 1))
    return res


# speedup vs baseline: 1.2600x; 1.2600x over previous
import functools
import math

import jax
import jax.numpy as jnp
from jax.experimental import pallas as pl
from jax.experimental.pallas import tpu as pltpu

_DT = jnp.bfloat16
_F8 = jnp.float8_e4m3fn

_TRANS_A = (((0,), (0,)), ((), ()))
_TRANS_B = (((1,), (1,)), ((), ()))


def _lap_kernel(ei_ref, lap_ref):
    e = ei_ref.shape[1]
    n = lap_ref.shape[0]
    ids = jax.lax.broadcasted_iota(jnp.int32, (n, e), 0)
    ert = (ids == ei_ref[0:1, :]).astype(_F8)
    ect = (ids == ei_ref[1:2, :]).astype(_F8)
    a = jax.lax.dot_general(ert, ect, _TRANS_B,
                            preferred_element_type=jnp.float32)
    deg = jnp.sum(a, axis=1, keepdims=True)
    rows = jax.lax.broadcasted_iota(jnp.int32, (n, n), 0)
    cols = jax.lax.broadcasted_iota(jnp.int32, (n, n), 1)
    lap_ref[...] = jnp.where(rows == cols, deg - a, -a)


def _compose_g_kernel(c_ref, lap_ref, g0_ref, g1_ref):
    lap = lap_ref[...]
    n = lap.shape[0]
    rows = jax.lax.broadcasted_iota(jnp.int32, (n, n), 0)
    cols = jax.lax.broadcasted_iota(jnp.int32, (n, n), 1)
    eye = rows == cols
    off = jnp.where(eye, 0.0, lap).astype(_DT)
    ld_all = jnp.sum(jnp.where(eye, lap, 0.0), axis=1, keepdims=True)

    def compose(i, g_ref):
        h = c_ref[i, 0]
        alpha = c_ref[i, 1]
        ld = ld_all - alpha
        hld = h * ld
        den = 1.0 / (hld * hld + 1.0)
        tlr = hld * den
        tli = -den
        dr = (-h) * tlr
        di = (-h) * tli

        hlm = h * jnp.where(eye, lap - alpha, lap)
        br = (tlr * hlm + jnp.where(eye, tli, 0.0)).astype(_DT)
        bi = (tli * hlm - jnp.where(eye, tlr, 0.0)).astype(_DT)

        def jmul(ur, ui):
            sr = (dr * ur - di * ui).astype(_DT)
            si = (dr * ui + di * ur).astype(_DT)
            return (jax.lax.dot_general(off, sr, _TRANS_A,
                                        preferred_element_type=jnp.float32),
                    jax.lax.dot_general(off, si, _TRANS_A,
                                        preferred_element_type=jnp.float32))

        jbr, jbi = jmul(br, bi)
        mr_f, mi_f = jmul((jbr + br).astype(_DT), (jbi + bi).astype(_DT))
        mr = (mr_f + br).astype(_DT)
        mi = (mi_f + bi).astype(_DT)

        ms = (mr.astype(jnp.float32) + mi.astype(jnp.float32)).astype(_DT)
        t1 = jnp.dot(mr, mr, preferred_element_type=jnp.float32)
        t2 = jnp.dot(mi, mi, preferred_element_type=jnp.float32)
        t3 = jnp.dot(ms, ms, preferred_element_type=jnp.float32)

        g = (2.0 * (c_ref[i, 3] * mr.astype(jnp.float32)
                    - c_ref[i, 4] * mi.astype(jnp.float32))
             + 2.0 * (c_ref[i, 5] * (t1 - t2)
                      - c_ref[i, 6] * (t3 - t1 - t2)))
        g_ref[...] = (g + jnp.where(eye, c_ref[i, 2], 0.0)).astype(g_ref.dtype)

    compose(0, g0_ref)
    compose(1, g1_ref)


def _apply_convs_kernel(g0_ref, g1_ref, w_ref, x_ref, out_ref, acc_ref):
    x = x_ref[...].astype(_DT)
    hid = jnp.dot(g0_ref[...], x, preferred_element_type=jnp.float32)
    hid = jnp.maximum(hid, 0.0).astype(_DT)
    o = jnp.dot(g1_ref[...], hid, preferred_element_type=jnp.float32)
    o = jnp.maximum(o, 0.0)
    out_ref[...] = o.astype(out_ref.dtype)

    @pl.when(pl.program_id(0) == 0)
    def _():
        acc_ref[...] = jnp.zeros_like(acc_ref)

    acc_ref[...] += jax.lax.dot_general(
        w_ref[...], o, (((0,), (1,)), ((), ())),
        preferred_element_type=jnp.float32)


def _epilogue_kernel(sacc_ref, pw_ref, lb_ref, out_ref, lw_ref, res_ref,
                     wv_ref, *, kk):
    i = pl.program_id(0)
    n = wv_ref.shape[1]

    @pl.when(i == 0)
    def _():
        pw = pw_ref[...]
        inv_norm = 1.0 / jnp.sqrt(jnp.sum(pw * pw))
        score = jnp.tanh(sacc_ref[...] * inv_norm)
        b = pltpu.bitcast(score, jnp.int32)
        key = jnp.where(b < 0, jnp.int32(-2147483648) - b, b)
        top = jnp.int32(0x3F800000)

        def body(_, carry):
            lo, hi = carry
            mid = lo + (hi - lo + 1) // 2
            go = jnp.sum((key >= mid).astype(jnp.int32)) >= kk
            return (jnp.where(go, mid, lo), jnp.where(go, hi, mid - 1))

        lo, _ = jax.lax.fori_loop(0, 32, body, (-top, top))

        c_gt = jnp.sum((key > lo).astype(jnp.int32))
        tie = key == lo
        ii = jax.lax.broadcasted_iota(jnp.int32, (n, n), 0)
        jj = jax.lax.broadcasted_iota(jnp.int32, (n, n), 1)
        tri = (ii <= jj).astype(_DT)
        pref = jnp.dot(tie.astype(_DT), tri,
                       preferred_element_type=jnp.float32)
        sel = (key > lo) | (tie & (pref <= (kk - c_gt).astype(jnp.float32)))
        wv_ref[...] = jnp.where(sel, score, 0.0)

    vt = jnp.dot(wv_ref[...].astype(_DT), out_ref[...],
                 preferred_element_type=jnp.float32) * (1.0 / kk)
    ft = jax.lax.dot_general(vt.astype(_DT), lw_ref[...].astype(_DT),
                             _TRANS_B, preferred_element_type=jnp.float32)

    @pl.when(i == 0)
    def _():
        res_ref[...] = lb_ref[...]

    res_ref[...] += ft


def kernel(x, edge_index, batch,
           conv0_h, conv0_alpha, conv0_c0, conv0_cjr, conv0_cji,
           conv1_h, conv1_alpha, conv1_c0, conv1_cjr, conv1_cji,
           pool_w, lin_w, lin_b):
    del batch
    n, f = x.shape
    e = edge_index.shape[1]
    nout = lin_w.shape[0]

    lap = pl.pallas_call(
        _lap_kernel,
        out_shape=jax.ShapeDtypeStruct((n, n), jnp.float32),
        in_specs=[pl.BlockSpec((2, e), lambda: (0, 0))],
        out_specs=pl.BlockSpec((n, n), lambda: (0, 0)),
    )(edge_index)

    cvec = jnp.stack([
        jnp.stack([conv0_h, conv0_alpha, conv0_c0, conv0_cjr[0],
                   conv0_cji[0], conv0_cjr[1], conv0_cji[1]]),
        jnp.stack([conv1_h, conv1_alpha, conv1_c0, conv1_cjr[0],
                   conv1_cji[0], conv1_cjr[1], conv1_cji[1]]),
    ]).astype(jnp.float32)

    g0, g1 = pl.pallas_call(
        _compose_g_kernel,
        out_shape=[jax.ShapeDtypeStruct((n, n), _DT),
                   jax.ShapeDtypeStruct((n, n), _DT)],
        in_specs=[pl.BlockSpec(memory_space=pltpu.MemorySpace.SMEM),
                  pl.BlockSpec((n, n), lambda: (0, 0))],
        out_specs=[pl.BlockSpec((n, n), lambda: (0, 0)),
                   pl.BlockSpec((n, n), lambda: (0, 0))],
    )(cvec, lap)

    tf = min(512, f)
    out, sacc = pl.pallas_call(
        _apply_convs_kernel,
        out_shape=[jax.ShapeDtypeStruct((n, f), _DT),
                   jax.ShapeDtypeStruct((1, n), jnp.float32)],
        grid=(f // tf,),
        in_specs=[pl.BlockSpec((n, n), lambda i: (0, 0)),
                  pl.BlockSpec((n, n), lambda i: (0, 0)),
                  pl.BlockSpec((tf, 1), lambda i: (i, 0)),
                  pl.BlockSpec((n, tf), lambda i: (0, i))],
        out_specs=[pl.BlockSpec((n, tf), lambda i: (0, i)),
                   pl.BlockSpec((1, n), lambda i: (0, 0))],
        compiler_params=pltpu.CompilerParams(
            dimension_semantics=("arbitrary",)),
    )(g0, g1, pool_w.reshape(f, 1), x)

    kk = int(math.ceil(0.9 * n))
    res = pl.pallas_call(
        functools.partial(_epilogue_kernel, kk=kk),
        out_shape=jax.ShapeDtypeStruct((1, nout), jnp.float32),
        grid=(f // tf,),
        in_specs=[pl.BlockSpec((1, n), lambda i: (0, 0)),
                  pl.BlockSpec((1, f), lambda i: (0, 0)),
                  pl.BlockSpec((1, nout), lambda i: (0, 0)),
                  pl.BlockSpec((n, tf), lambda i: (0, i)),
                  pl.BlockSpec((nout, tf), lambda i: (0, i))],
        out_specs=pl.BlockSpec((1, nout), lambda i: (0, 0)),
        scratch_shapes=[pltpu.VMEM((1, n), jnp.float32)],
        compiler_params=pltpu.CompilerParams(
            dimension_semantics=("arbitrary",)),
    )(sacc, pool_w.reshape(1, f), lin_b.reshape(1, nout), out, lin_w)
    return res


# speedup vs baseline: 1.3447x; 1.0672x over previous
import functools
import math

import jax
import jax.numpy as jnp
from jax.experimental import pallas as pl
from jax.experimental.pallas import tpu as pltpu

_DT = jnp.bfloat16
_F8 = jnp.float8_e4m3fn

_TRANS_A = (((0,), (0,)), ((), ()))
_TRANS_B = (((1,), (1,)), ((), ()))


def _lap_kernel(ei_ref, lap_ref):
    e = ei_ref.shape[1]
    n = lap_ref.shape[0]
    ids = jax.lax.broadcasted_iota(jnp.int32, (n, e), 0)
    ert = (ids == ei_ref[0:1, :]).astype(_F8)
    ect = (ids == ei_ref[1:2, :]).astype(_F8)
    a = jax.lax.dot_general(ert, ect, _TRANS_B,
                            preferred_element_type=jnp.float32)
    deg = jnp.sum(a, axis=1, keepdims=True)
    rows = jax.lax.broadcasted_iota(jnp.int32, (n, n), 0)
    cols = jax.lax.broadcasted_iota(jnp.int32, (n, n), 1)
    lap_ref[...] = jnp.where(rows == cols, deg - a, -a)


def _compose_g_kernel(c_ref, lap_ref, g0_ref, g1_ref):
    lap = lap_ref[...]
    n = lap.shape[0]
    rows = jax.lax.broadcasted_iota(jnp.int32, (n, n), 0)
    cols = jax.lax.broadcasted_iota(jnp.int32, (n, n), 1)
    eye = rows == cols
    off = jnp.where(eye, 0.0, lap).astype(_DT)
    ld_all = jnp.sum(jnp.where(eye, lap, 0.0), axis=1, keepdims=True)

    def compose(i, g_ref):
        h = c_ref[i, 0]
        alpha = c_ref[i, 1]
        ld = ld_all - alpha
        hld = h * ld
        den = 1.0 / (hld * hld + 1.0)
        tlr = hld * den
        tli = -den
        dr = (-h) * tlr
        di = (-h) * tli

        hlm = h * jnp.where(eye, lap - alpha, lap)
        br = (tlr * hlm + jnp.where(eye, tli, 0.0)).astype(_DT)
        bi = (tli * hlm - jnp.where(eye, tlr, 0.0)).astype(_DT)

        def jmul(ur, ui):
            sr = (dr * ur - di * ui).astype(_DT)
            si = (dr * ui + di * ur).astype(_DT)
            return (jax.lax.dot_general(off, sr, _TRANS_A,
                                        preferred_element_type=jnp.float32),
                    jax.lax.dot_general(off, si, _TRANS_A,
                                        preferred_element_type=jnp.float32))

        jbr, jbi = jmul(br, bi)
        mr_f, mi_f = jmul((jbr + br).astype(_DT), (jbi + bi).astype(_DT))
        mr = (mr_f + br).astype(_DT)
        mi = (mi_f + bi).astype(_DT)

        ms = (mr.astype(jnp.float32) + mi.astype(jnp.float32)).astype(_DT)
        t1 = jnp.dot(mr, mr, preferred_element_type=jnp.float32)
        t2 = jnp.dot(mi, mi, preferred_element_type=jnp.float32)
        t3 = jnp.dot(ms, ms, preferred_element_type=jnp.float32)

        g = (2.0 * (c_ref[i, 3] * mr.astype(jnp.float32)
                    - c_ref[i, 4] * mi.astype(jnp.float32))
             + 2.0 * (c_ref[i, 5] * (t1 - t2)
                      - c_ref[i, 6] * (t3 - t1 - t2)))
        g_ref[...] = (g + jnp.where(eye, c_ref[i, 2], 0.0)).astype(g_ref.dtype)

    compose(0, g0_ref)
    compose(1, g1_ref)


def _apply_epilogue_kernel(g0_ref, g1_ref, pw_ref, lb_ref, lw_ref, x_ref,
                           res_ref, outs_ref, sacc_ref, wv_ref, *, tf, kk):
    i = pl.program_id(0)
    nf = pl.num_programs(0)
    n = wv_ref.shape[1]

    x = x_ref[...].astype(_DT)
    hid = jnp.dot(g0_ref[...], x, preferred_element_type=jnp.float32)
    hid = jnp.maximum(hid, 0.0).astype(_DT)
    o = jnp.dot(g1_ref[...], hid, preferred_element_type=jnp.float32)
    o = jnp.maximum(o, 0.0)
    outs_ref[:, pl.ds(pl.multiple_of(i * tf, tf), tf)] = o.astype(_DT)

    @pl.when(i == 0)
    def _():
        sacc_ref[...] = jnp.zeros_like(sacc_ref)

    wt = pw_ref[pl.ds(pl.multiple_of(i * tf, tf), tf), :]
    sacc_ref[...] += jax.lax.dot_general(
        wt, o, (((0,), (1,)), ((), ())),
        preferred_element_type=jnp.float32)

    @pl.when(i == nf - 1)
    def _():
        pw = pw_ref[...]
        ssq = jax.lax.dot_general(pw, pw, _TRANS_A,
                                  preferred_element_type=jnp.float32)
        inv_norm = 1.0 / jnp.sqrt(ssq[0, 0])
        score = jnp.tanh(sacc_ref[...] * inv_norm)
        b = pltpu.bitcast(score, jnp.int32)
        key = jnp.where(b < 0, jnp.int32(-2147483648) - b, b)
        top = jnp.int32(0x3F800000)

        def body(_, carry):
            lo, hi = carry
            mid = lo + (hi - lo + 1) // 2
            go = jnp.sum((key >= mid).astype(jnp.int32)) >= kk
            return (jnp.where(go, mid, lo), jnp.where(go, hi, mid - 1))

        lo, _ = jax.lax.fori_loop(0, 32, body, (-top, top))

        c_gt = jnp.sum((key > lo).astype(jnp.int32))
        tie = key == lo
        ii = jax.lax.broadcasted_iota(jnp.int32, (n, n), 0)
        jj = jax.lax.broadcasted_iota(jnp.int32, (n, n), 1)
        tri = (ii <= jj).astype(_DT)
        pref = jnp.dot(tie.astype(_DT), tri,
                       preferred_element_type=jnp.float32)
        sel = (key > lo) | (tie & (pref <= (kk - c_gt).astype(jnp.float32)))
        wv_ref[...] = jnp.where(sel, score, 0.0)

        vt = jnp.dot(wv_ref[...].astype(_DT), outs_ref[...],
                     preferred_element_type=jnp.float32) * (1.0 / kk)
        ft = jax.lax.dot_general(vt.astype(_DT), lw_ref[...].astype(_DT),
                                 _TRANS_B,
                                 preferred_element_type=jnp.float32)
        res_ref[...] = lb_ref[...] + ft


def kernel(x, edge_index, batch,
           conv0_h, conv0_alpha, conv0_c0, conv0_cjr, conv0_cji,
           conv1_h, conv1_alpha, conv1_c0, conv1_cjr, conv1_cji,
           pool_w, lin_w, lin_b):
    del batch
    n, f = x.shape
    e = edge_index.shape[1]
    nout = lin_w.shape[0]

    lap = pl.pallas_call(
        _lap_kernel,
        out_shape=jax.ShapeDtypeStruct((n, n), jnp.float32),
        in_specs=[pl.BlockSpec((2, e), lambda: (0, 0))],
        out_specs=pl.BlockSpec((n, n), lambda: (0, 0)),
    )(edge_index)

    cvec = jnp.stack([
        jnp.stack([conv0_h, conv0_alpha, conv0_c0, conv0_cjr[0],
                   conv0_cji[0], conv0_cjr[1], conv0_cji[1]]),
        jnp.stack([conv1_h, conv1_alpha, conv1_c0, conv1_cjr[0],
                   conv1_cji[0], conv1_cjr[1], conv1_cji[1]]),
    ]).astype(jnp.float32)

    g0, g1 = pl.pallas_call(
        _compose_g_kernel,
        out_shape=[jax.ShapeDtypeStruct((n, n), _DT),
                   jax.ShapeDtypeStruct((n, n), _DT)],
        in_specs=[pl.BlockSpec(memory_space=pltpu.MemorySpace.SMEM),
                  pl.BlockSpec((n, n), lambda: (0, 0))],
        out_specs=[pl.BlockSpec((n, n), lambda: (0, 0)),
                   pl.BlockSpec((n, n), lambda: (0, 0))],
    )(cvec, lap)

    tf = min(512, f)
    kk = int(math.ceil(0.9 * n))
    res = pl.pallas_call(
        functools.partial(_apply_epilogue_kernel, tf=tf, kk=kk),
        out_shape=jax.ShapeDtypeStruct((1, nout), jnp.float32),
        grid=(f // tf,),
        in_specs=[pl.BlockSpec((n, n), lambda i: (0, 0)),
                  pl.BlockSpec((n, n), lambda i: (0, 0)),
                  pl.BlockSpec((f, 1), lambda i: (0, 0)),
                  pl.BlockSpec((1, nout), lambda i: (0, 0)),
                  pl.BlockSpec((nout, f), lambda i: (0, 0)),
                  pl.BlockSpec((n, tf), lambda i: (0, i))],
        out_specs=pl.BlockSpec((1, nout), lambda i: (0, 0)),
        scratch_shapes=[pltpu.VMEM((n, f), _DT),
                        pltpu.VMEM((1, n), jnp.float32),
                        pltpu.VMEM((1, n), jnp.float32)],
        compiler_params=pltpu.CompilerParams(
            dimension_semantics=("arbitrary",)),
    )(g0, g1, pool_w.reshape(f, 1), lin_b.reshape(1, nout), lin_w, x)
    return res


# speedup vs baseline: 1.3572x; 1.0093x over previous
import functools
import math

import jax
import jax.numpy as jnp
from jax.experimental import pallas as pl
from jax.experimental.pallas import tpu as pltpu

_DT = jnp.bfloat16
_F8 = jnp.float8_e4m3fn

_TRANS_A = (((0,), (0,)), ((), ()))
_TRANS_B = (((1,), (1,)), ((), ()))


def _lap_kernel(ei_ref, lap_ref):
    e = ei_ref.shape[1]
    n = lap_ref.shape[0]
    ids = jax.lax.broadcasted_iota(jnp.int32, (n, e), 0)
    ert = (ids == ei_ref[0:1, :]).astype(_F8)
    ect = (ids == ei_ref[1:2, :]).astype(_F8)
    a = jax.lax.dot_general(ert, ect, _TRANS_B,
                            preferred_element_type=jnp.float32)
    deg = jnp.sum(a, axis=1, keepdims=True)
    rows = jax.lax.broadcasted_iota(jnp.int32, (n, n), 0)
    cols = jax.lax.broadcasted_iota(jnp.int32, (n, n), 1)
    lap_ref[...] = jnp.where(rows == cols, deg - a, -a)


def _compose_g_kernel(c_ref, lap_ref, g0_ref, g1_ref):
    lap = lap_ref[...]
    n = lap.shape[0]
    rows = jax.lax.broadcasted_iota(jnp.int32, (n, n), 0)
    cols = jax.lax.broadcasted_iota(jnp.int32, (n, n), 1)
    eye = rows == cols
    off = jnp.where(eye, 0.0, lap).astype(_DT)
    ld_all = jnp.sum(jnp.where(eye, lap, 0.0), axis=1, keepdims=True)

    def compose(i, g_ref):
        h = c_ref[i, 0]
        alpha = c_ref[i, 1]
        ld = ld_all - alpha
        hld = h * ld
        den = 1.0 / (hld * hld + 1.0)
        tlr = hld * den
        tli = -den
        dr = (-h) * tlr
        di = (-h) * tli

        hlm = h * jnp.where(eye, lap - alpha, lap)
        br = (tlr * hlm + jnp.where(eye, tli, 0.0)).astype(_DT)
        bi = (tli * hlm - jnp.where(eye, tlr, 0.0)).astype(_DT)

        def jmul(ur, ui):
            sr = (dr * ur - di * ui).astype(_DT)
            si = (dr * ui + di * ur).astype(_DT)
            return (jax.lax.dot_general(off, sr, _TRANS_A,
                                        preferred_element_type=jnp.float32),
                    jax.lax.dot_general(off, si, _TRANS_A,
                                        preferred_element_type=jnp.float32))

        jbr, jbi = jmul(br, bi)
        mr_f, mi_f = jmul((jbr + br).astype(_DT), (jbi + bi).astype(_DT))
        mr = (mr_f + br).astype(_DT)
        mi = (mi_f + bi).astype(_DT)

        ms = (mr.astype(jnp.float32) + mi.astype(jnp.float32)).astype(_DT)
        t1 = jnp.dot(mr, mr, preferred_element_type=jnp.float32)
        t2 = jnp.dot(mi, mi, preferred_element_type=jnp.float32)
        t3 = jnp.dot(ms, ms, preferred_element_type=jnp.float32)

        g = (2.0 * (c_ref[i, 3] * mr.astype(jnp.float32)
                    - c_ref[i, 4] * mi.astype(jnp.float32))
             + 2.0 * (c_ref[i, 5] * (t1 - t2)
                      - c_ref[i, 6] * (t3 - t1 - t2)))
        g_ref[...] = (g + jnp.where(eye, c_ref[i, 2], 0.0)).astype(g_ref.dtype)

    compose(0, g0_ref)
    compose(1, g1_ref)


def _apply_epilogue_kernel(g0_ref, g1_ref, pw_ref, lb_ref, lw_ref, x_ref,
                           res_ref, outs_ref, sacc_ref, wv_ref, *, tf, kk):
    i = pl.program_id(0)
    nf = pl.num_programs(0)
    n = wv_ref.shape[1]

    x = x_ref[...].astype(_DT)
    hid = jnp.dot(g0_ref[...], x, preferred_element_type=jnp.float32)
    hid = jnp.maximum(hid, 0.0).astype(_DT)
    o = jnp.dot(g1_ref[...], hid, preferred_element_type=jnp.float32)
    o = jnp.maximum(o, 0.0)
    outs_ref[:, pl.ds(pl.multiple_of(i * tf, tf), tf)] = o.astype(_DT)

    @pl.when(i == 0)
    def _():
        sacc_ref[...] = jnp.zeros_like(sacc_ref)

    wt = pw_ref[pl.ds(pl.multiple_of(i * tf, tf), tf), :]
    sacc_ref[...] += jax.lax.dot_general(
        wt, o, (((0,), (1,)), ((), ())),
        preferred_element_type=jnp.float32)

    @pl.when(i == nf - 1)
    def _():
        pw = pw_ref[...]
        ssq = jax.lax.dot_general(pw, pw, _TRANS_A,
                                  preferred_element_type=jnp.float32)
        inv_norm = 1.0 / jnp.sqrt(ssq[0, 0])
        score = jnp.tanh(sacc_ref[...] * inv_norm)
        b = pltpu.bitcast(score, jnp.int32)
        key = jnp.where(b < 0, jnp.int32(-2147483648) - b, b)
        top = jnp.int32(0x3F800000)

        def body(_, carry):
            lo, hi = carry
            mid = lo + (hi - lo + 1) // 2
            go = jnp.sum((key >= mid).astype(jnp.int32)) >= kk
            return (jnp.where(go, mid, lo), jnp.where(go, hi, mid - 1))

        lo, _ = jax.lax.fori_loop(0, 32, body, (-top, top))

        c_gt = jnp.sum((key > lo).astype(jnp.int32))
        tie = key == lo
        ii = jax.lax.broadcasted_iota(jnp.int32, (n, n), 0)
        jj = jax.lax.broadcasted_iota(jnp.int32, (n, n), 1)
        tri = (ii <= jj).astype(_DT)
        pref = jnp.dot(tie.astype(_DT), tri,
                       preferred_element_type=jnp.float32)
        sel = (key > lo) | (tie & (pref <= (kk - c_gt).astype(jnp.float32)))
        wv_ref[...] = jnp.where(sel, score, 0.0)

        vt = jnp.dot(wv_ref[...].astype(_DT), outs_ref[...],
                     preferred_element_type=jnp.float32) * (1.0 / kk)
        ft = jax.lax.dot_general(vt.astype(_DT), lw_ref[...].astype(_DT),
                                 _TRANS_B,
                                 preferred_element_type=jnp.float32)
        res_ref[...] = lb_ref[...] + ft


def kernel(x, edge_index, batch,
           conv0_h, conv0_alpha, conv0_c0, conv0_cjr, conv0_cji,
           conv1_h, conv1_alpha, conv1_c0, conv1_cjr, conv1_cji,
           pool_w, lin_w, lin_b):
    del batch
    n, f = x.shape
    e = edge_index.shape[1]
    nout = lin_w.shape[0]

    lap = pl.pallas_call(
        _lap_kernel,
        out_shape=jax.ShapeDtypeStruct((n, n), jnp.float32),
        in_specs=[pl.BlockSpec((2, e), lambda: (0, 0))],
        out_specs=pl.BlockSpec((n, n), lambda: (0, 0)),
    )(edge_index)

    cvec = jnp.stack([
        jnp.stack([conv0_h, conv0_alpha, conv0_c0, conv0_cjr[0],
                   conv0_cji[0], conv0_cjr[1], conv0_cji[1]]),
        jnp.stack([conv1_h, conv1_alpha, conv1_c0, conv1_cjr[0],
                   conv1_cji[0], conv1_cjr[1], conv1_cji[1]]),
    ]).astype(jnp.float32)

    g0, g1 = pl.pallas_call(
        _compose_g_kernel,
        out_shape=[jax.ShapeDtypeStruct((n, n), _DT),
                   jax.ShapeDtypeStruct((n, n), _DT)],
        in_specs=[pl.BlockSpec(memory_space=pltpu.MemorySpace.SMEM),
                  pl.BlockSpec((n, n), lambda: (0, 0))],
        out_specs=[pl.BlockSpec((n, n), lambda: (0, 0)),
                   pl.BlockSpec((n, n), lambda: (0, 0))],
    )(cvec, lap)

    tf = min(1024, f)
    kk = int(math.ceil(0.9 * n))
    res = pl.pallas_call(
        functools.partial(_apply_epilogue_kernel, tf=tf, kk=kk),
        out_shape=jax.ShapeDtypeStruct((1, nout), jnp.float32),
        grid=(f // tf,),
        in_specs=[pl.BlockSpec((n, n), lambda i: (0, 0)),
                  pl.BlockSpec((n, n), lambda i: (0, 0)),
                  pl.BlockSpec((f, 1), lambda i: (0, 0)),
                  pl.BlockSpec((1, nout), lambda i: (0, 0)),
                  pl.BlockSpec((nout, f), lambda i: (0, 0)),
                  pl.BlockSpec((n, tf), lambda i: (0, i))],
        out_specs=pl.BlockSpec((1, nout), lambda i: (0, 0)),
        scratch_shapes=[pltpu.VMEM((n, f), _DT),
                        pltpu.VMEM((1, n), jnp.float32),
                        pltpu.VMEM((1, n), jnp.float32)],
        compiler_params=pltpu.CompilerParams(
            dimension_semantics=("arbitrary",)),
    )(g0, g1, pool_w.reshape(f, 1), lin_b.reshape(1, nout), lin_w, x)
    return res


# speedup vs baseline: 1.4088x; 1.0380x over previous
import functools
import math

import jax
import jax.numpy as jnp
from jax.experimental import pallas as pl
from jax.experimental.pallas import tpu as pltpu

_DT = jnp.bfloat16
_F8 = jnp.float8_e4m3fn

_TRANS_A = (((0,), (0,)), ((), ()))
_TRANS_B = (((1,), (1,)), ((), ()))


def _compose_g_kernel(c_ref, ei_ref, g0_ref, g1_ref):
    e = ei_ref.shape[1]
    n = g0_ref.shape[0]
    ids = jax.lax.broadcasted_iota(jnp.int32, (n, e), 0)
    ert = (ids == ei_ref[0:1, :]).astype(_F8)
    ect = (ids == ei_ref[1:2, :]).astype(_F8)
    a = jax.lax.dot_general(ert, ect, _TRANS_B,
                            preferred_element_type=jnp.float32)
    deg = jnp.sum(a, axis=1, keepdims=True)
    rows = jax.lax.broadcasted_iota(jnp.int32, (n, n), 0)
    cols = jax.lax.broadcasted_iota(jnp.int32, (n, n), 1)
    eye = rows == cols
    lap = jnp.where(eye, deg - a, -a)
    off = jnp.where(eye, 0.0, lap).astype(_DT)
    ld_all = jnp.sum(jnp.where(eye, lap, 0.0), axis=1, keepdims=True)

    def compose(i, g_ref):
        h = c_ref[i, 0]
        alpha = c_ref[i, 1]
        ld = ld_all - alpha
        hld = h * ld
        den = 1.0 / (hld * hld + 1.0)
        tlr = hld * den
        tli = -den
        dr = (-h) * tlr
        di = (-h) * tli

        hlm = h * jnp.where(eye, lap - alpha, lap)
        br = (tlr * hlm + jnp.where(eye, tli, 0.0)).astype(_DT)
        bi = (tli * hlm - jnp.where(eye, tlr, 0.0)).astype(_DT)

        def jmul(ur, ui):
            sr = (dr * ur - di * ui).astype(_DT)
            si = (dr * ui + di * ur).astype(_DT)
            return (jax.lax.dot_general(off, sr, _TRANS_A,
                                        preferred_element_type=jnp.float32),
                    jax.lax.dot_general(off, si, _TRANS_A,
                                        preferred_element_type=jnp.float32))

        jbr, jbi = jmul(br, bi)
        mr_f, mi_f = jmul((jbr + br).astype(_DT), (jbi + bi).astype(_DT))
        mr = (mr_f + br).astype(_DT)
        mi = (mi_f + bi).astype(_DT)

        ms = (mr.astype(jnp.float32) + mi.astype(jnp.float32)).astype(_DT)
        t1 = jnp.dot(mr, mr, preferred_element_type=jnp.float32)
        t2 = jnp.dot(mi, mi, preferred_element_type=jnp.float32)
        t3 = jnp.dot(ms, ms, preferred_element_type=jnp.float32)

        g = (2.0 * (c_ref[i, 3] * mr.astype(jnp.float32)
                    - c_ref[i, 4] * mi.astype(jnp.float32))
             + 2.0 * (c_ref[i, 5] * (t1 - t2)
                      - c_ref[i, 6] * (t3 - t1 - t2)))
        g_ref[...] = (g + jnp.where(eye, c_ref[i, 2], 0.0)).astype(g_ref.dtype)

    compose(0, g0_ref)
    compose(1, g1_ref)


def _apply_epilogue_kernel(g0_ref, g1_ref, pw_ref, lb_ref, lw_ref, x_ref,
                           res_ref, outs_ref, sacc_ref, wv_ref, *, tf, kk):
    i = pl.program_id(0)
    nf = pl.num_programs(0)
    n = wv_ref.shape[1]

    x = x_ref[...].astype(_DT)
    hid = jnp.dot(g0_ref[...], x, preferred_element_type=jnp.float32)
    hid = jnp.maximum(hid, 0.0).astype(_DT)
    o = jnp.dot(g1_ref[...], hid, preferred_element_type=jnp.float32)
    o = jnp.maximum(o, 0.0)
    outs_ref[:, pl.ds(pl.multiple_of(i * tf, tf), tf)] = o.astype(_DT)

    @pl.when(i == 0)
    def _():
        sacc_ref[...] = jnp.zeros_like(sacc_ref)

    wt = pw_ref[pl.ds(pl.multiple_of(i * tf, tf), tf), :]
    sacc_ref[...] += jax.lax.dot_general(
        wt, o, (((0,), (1,)), ((), ())),
        preferred_element_type=jnp.float32)

    @pl.when(i == nf - 1)
    def _():
        pw = pw_ref[...]
        ssq = jax.lax.dot_general(pw, pw, _TRANS_A,
                                  preferred_element_type=jnp.float32)
        inv_norm = 1.0 / jnp.sqrt(ssq[0, 0])
        score = jnp.tanh(sacc_ref[...] * inv_norm)
        b = pltpu.bitcast(score, jnp.int32)
        key = jnp.where(b < 0, jnp.int32(-2147483648) - b, b)
        top = jnp.int32(0x3F800000)

        def body(_, carry):
            lo, hi = carry
            mid = lo + (hi - lo + 1) // 2
            go = jnp.sum((key >= mid).astype(jnp.int32)) >= kk
            return (jnp.where(go, mid, lo), jnp.where(go, hi, mid - 1))

        lo, _ = jax.lax.fori_loop(0, 32, body, (-top, top))

        c_gt = jnp.sum((key > lo).astype(jnp.int32))
        tie = key == lo
        ii = jax.lax.broadcasted_iota(jnp.int32, (n, n), 0)
        jj = jax.lax.broadcasted_iota(jnp.int32, (n, n), 1)
        tri = (ii <= jj).astype(_DT)
        pref = jnp.dot(tie.astype(_DT), tri,
                       preferred_element_type=jnp.float32)
        sel = (key > lo) | (tie & (pref <= (kk - c_gt).astype(jnp.float32)))
        wv_ref[...] = jnp.where(sel, score, 0.0)

        vt = jnp.dot(wv_ref[...].astype(_DT), outs_ref[...],
                     preferred_element_type=jnp.float32) * (1.0 / kk)
        ft = jax.lax.dot_general(vt.astype(_DT), lw_ref[...].astype(_DT),
                                 _TRANS_B,
                                 preferred_element_type=jnp.float32)
        res_ref[...] = lb_ref[...] + ft


def kernel(x, edge_index, batch,
           conv0_h, conv0_alpha, conv0_c0, conv0_cjr, conv0_cji,
           conv1_h, conv1_alpha, conv1_c0, conv1_cjr, conv1_cji,
           pool_w, lin_w, lin_b):
    del batch
    n, f = x.shape
    e = edge_index.shape[1]
    nout = lin_w.shape[0]

    cvec = jnp.stack([
        jnp.stack([conv0_h, conv0_alpha, conv0_c0, conv0_cjr[0],
                   conv0_cji[0], conv0_cjr[1], conv0_cji[1]]),
        jnp.stack([conv1_h, conv1_alpha, conv1_c0, conv1_cjr[0],
                   conv1_cji[0], conv1_cjr[1], conv1_cji[1]]),
    ]).astype(jnp.float32)

    g0, g1 = pl.pallas_call(
        _compose_g_kernel,
        out_shape=[jax.ShapeDtypeStruct((n, n), _DT),
                   jax.ShapeDtypeStruct((n, n), _DT)],
        in_specs=[pl.BlockSpec(memory_space=pltpu.MemorySpace.SMEM),
                  pl.BlockSpec((2, e), lambda: (0, 0))],
        out_specs=[pl.BlockSpec((n, n), lambda: (0, 0)),
                   pl.BlockSpec((n, n), lambda: (0, 0))],
    )(cvec, edge_index)

    tf = min(1024, f)
    kk = int(math.ceil(0.9 * n))
    res = pl.pallas_call(
        functools.partial(_apply_epilogue_kernel, tf=tf, kk=kk),
        out_shape=jax.ShapeDtypeStruct((1, nout), jnp.float32),
        grid=(f // tf,),
        in_specs=[pl.BlockSpec((n, n), lambda i: (0, 0)),
                  pl.BlockSpec((n, n), lambda i: (0, 0)),
                  pl.BlockSpec((f, 1), lambda i: (0, 0)),
                  pl.BlockSpec((1, nout), lambda i: (0, 0)),
                  pl.BlockSpec((nout, f), lambda i: (0, 0)),
                  pl.BlockSpec((n, tf), lambda i: (0, i))],
        out_specs=pl.BlockSpec((1, nout), lambda i: (0, 0)),
        scratch_shapes=[pltpu.VMEM((n, f), _DT),
                        pltpu.VMEM((1, n), jnp.float32),
                        pltpu.VMEM((1, n), jnp.float32)],
        compiler_params=pltpu.CompilerParams(
            dimension_semantics=("arbitrary",)),
    )(g0, g1, pool_w.reshape(f, 1), lin_b.reshape(1, nout), lin_w, x)
    return res
